# Initial kernel scaffold; baseline (speedup 1.0000x reference)
import jax
import jax.numpy as jnp
from jax.experimental import pallas as pl


def kernel(x_user, x_event, W_u0, b_u0, W_e0, b_e0, W_u1, b_u1, W_e1, b_e1, W_ih, W_hh, b_ih, b_hh, W_lin, b_lin, start, user_embedding, edge_index, ids, sequence, batch_size):
    raise NotImplementedError("write your pallas kernel here")


# speedup vs baseline: 2.1560x; 2.1560x over previous
import functools

import jax
import jax.numpy as jnp
from jax import lax
from jax.experimental import pallas as pl
from jax.experimental.pallas import tpu as pltpu
from jax.experimental.pallas import tpu_sc as plsc

NU = 10000
NE = 10000
D = 256
E = 160000
B = 64
L = 20
TP = 10240
EP = 163840
EPT = EP // 16
CH = 64
NCH = EPT // CH
H = 128
TPAD = 24
VP = 10240
NEG = -1e9

def _mesh():
    return plsc.VectorSubcoreMesh(core_axis_name="c", subcore_axis_name="s",
                                  num_cores=2, num_subcores=16)


def _dense0_body(xu_ref, xe_ref, wu_ref, we_ref, bu_ref, be_ref,
                 us_ref, ev_ref):
    us = jnp.dot(xu_ref[...], wu_ref[...],
                 preferred_element_type=jnp.float32) + bu_ref[...]
    ev = jnp.dot(xe_ref[...], we_ref[...],
                 preferred_element_type=jnp.float32) + be_ref[...]
    us_ref[0] = us[:, :H]
    us_ref[1] = us[:, H:]
    ev_ref[0] = ev[:, :H]
    ev_ref[1] = ev[:, H:]


def _dense0(x_user, x_event, wu_t, we_t, bu, be):
    blk = 1000
    return pl.pallas_call(
        _dense0_body,
        grid=(NU // blk,),
        in_specs=[
            pl.BlockSpec((blk, D), lambda r: (r, 0)),
            pl.BlockSpec((blk, D), lambda r: (r, 0)),
            pl.BlockSpec((D, D), lambda r: (0, 0)),
            pl.BlockSpec((D, D), lambda r: (0, 0)),
            pl.BlockSpec((1, D), lambda r: (0, 0)),
            pl.BlockSpec((1, D), lambda r: (0, 0)),
        ],
        out_specs=[
            pl.BlockSpec((2, blk, H), lambda r: (0, r, 0)),
            pl.BlockSpec((2, blk, H), lambda r: (0, r, 0)),
        ],
        out_shape=[
            jax.ShapeDtypeStruct((2, TP, H), jnp.float32),
            jax.ShapeDtypeStruct((2, TP, H), jnp.float32),
        ],
    )(x_user, x_event, wu_t, we_t, bu, be)


def _dense1_body(agg_ref, us0_ref, deg_ref, w_ref, b_ref, out_ref):
    agg = jnp.concatenate([agg_ref[0], agg_ref[1]], axis=1)
    us0 = jnp.concatenate([us0_ref[0], us0_ref[1]], axis=1)
    new_u = (agg + us0) * (1.0 / (deg_ref[...] + 1.0))
    us1 = jnp.dot(new_u, w_ref[...],
                  preferred_element_type=jnp.float32) + b_ref[...]
    out_ref[0] = us1[:, :H]
    out_ref[1] = us1[:, H:]


def _dense1(agg_u, us0s, deg_u_col, w1_t, b1):
    blk = 1000
    return pl.pallas_call(
        _dense1_body,
        grid=(NU // blk,),
        in_specs=[
            pl.BlockSpec((2, blk, H), lambda r: (0, r, 0)),
            pl.BlockSpec((2, blk, H), lambda r: (0, r, 0)),
            pl.BlockSpec((blk, 1), lambda r: (r, 0)),
            pl.BlockSpec((D, D), lambda r: (0, 0)),
            pl.BlockSpec((1, D), lambda r: (0, 0)),
        ],
        out_specs=pl.BlockSpec((2, blk, H), lambda r: (0, r, 0)),
        out_shape=jax.ShapeDtypeStruct((2, TP, H), jnp.float32),
    )(agg_u, us0s, deg_u_col, w1_t, b1)


NQ = 4
QCH = NCH // NQ
QE = EPT // NQ


def _sc_edge_pass(table_hbm, gsrc1d, widx2d, s, coff, shared,
                  idxoff, widx_t2, rows0, rows1, sem0, sem1, hist=None):
    def g_start(j, buf, sem):
        off = pl.multiple_of(j * CH, CH)
        pltpu.async_copy(table_hbm.at[idxoff.at[pl.ds(off, CH)]], buf, sem)

    def g_wait(buf, sem):
        pltpu.make_async_copy(
            table_hbm.at[idxoff.at[pl.ds(0, CH)]], buf, sem).wait()

    for q in range(NQ):
        pltpu.sync_copy(gsrc1d.at[pl.ds(s * EPT + q * QE, QE)], idxoff)
        _offset_idx(idxoff, coff, QE)
        pltpu.sync_copy(widx2d.at[pl.ds(s * NCH + q * QCH, QCH)], widx_t2)
        if hist is not None:
            c, src2d_hbm, hsrc_t2, hist_ref, ones = hist
            pltpu.sync_copy(src2d_hbm.at[pl.ds(s * NCH + q * QCH, QCH)],
                            hsrc_t2)

        def do_chunk(j, buf):
            pltpu.sync_copy(buf, shared.at[widx_t2.at[j]], add=True)
            if hist is not None:
                @pl.when(c == 0)
                def _():
                    pltpu.sync_copy(ones, hist_ref.at[widx_t2.at[j]],
                                    add=True)

                @pl.when(c == 1)
                def _():
                    pltpu.sync_copy(ones, hist_ref.at[hsrc_t2.at[j]],
                                    add=True)

        g_start(0, rows0, sem0)

        def body(i, carry):
            j0 = i * 2
            j1 = j0 + 1
            g_wait(rows0, sem0)
            g_start(j1, rows1, sem1)
            do_chunk(j0, rows0)
            g_wait(rows1, sem1)

            @pl.when(i < QCH // 2 - 1)
            def _():
                g_start(j0 + 2, rows0, sem0)

            do_chunk(j1, rows1)
            return carry

        lax.fori_loop(0, QCH // 2, body, 0)


def _offset_idx(idx_ref, coff, n):
    def body(i, carry):
        off = pl.multiple_of(i * 16, 16)
        idx_ref[pl.ds(off, 16)] = idx_ref[pl.ds(off, 16)] + coff
        return carry

    lax.fori_loop(0, n // 16, body, 0)


def _sc1_body(ev0f, us0f, src1d, dst1d, src2d, dst2d, zeros2d, zeros1d,
              aggu_hbm, agge0_hbm, degu_hbm, dege_hbm,
              idxoff, widx_t2, hsrc_t2, rows0, rows1, ones, shared, hist,
              sem0, sem1):
    c = lax.axis_index("c")
    s = lax.axis_index("s")
    coff = c * TP

    for li in range(CH // 16):
        ones[pl.ds(li * 16, 16)] = jnp.full((16,), 1.0, jnp.float32)

    pltpu.sync_copy(zeros2d.at[pl.ds(s * 640, 640)],
                    shared.at[pl.ds(s * 640, 640)])

    @pl.when(s == 0)
    def _():
        pltpu.sync_copy(zeros1d, hist)

    plsc.subcore_barrier()

    _sc_edge_pass(ev0f, src1d, dst2d, s, coff, shared,
                  idxoff, widx_t2, rows0, rows1, sem0, sem1,
                  hist=(c, src2d, hsrc_t2, hist, ones))
    plsc.subcore_barrier()
    pltpu.sync_copy(shared.at[pl.ds(s * 640, 640)],
                    aggu_hbm.at[pl.ds(coff + s * 640, 640)])

    @pl.when((c == 0) & (s == 0))
    def _():
        pltpu.sync_copy(hist, degu_hbm)

    @pl.when((c == 1) & (s == 0))
    def _():
        pltpu.sync_copy(hist, dege_hbm)

    plsc.subcore_barrier()
    pltpu.sync_copy(zeros2d.at[pl.ds(s * 640, 640)],
                    shared.at[pl.ds(s * 640, 640)])
    plsc.subcore_barrier()

    _sc_edge_pass(us0f, dst1d, src2d, s, coff, shared,
                  idxoff, widx_t2, rows0, rows1, sem0, sem1)
    plsc.subcore_barrier()
    pltpu.sync_copy(shared.at[pl.ds(s * 640, 640)],
                    agge0_hbm.at[pl.ds(coff + s * 640, 640)])


def _sc1(ev0f, us0f, src1d, dst1d, src2d, dst2d, zeros2d, zeros1d):
    return pl.kernel(
        _sc1_body,
        out_type=[
            jax.ShapeDtypeStruct((2 * TP, H), jnp.float32),
            jax.ShapeDtypeStruct((2 * TP, H), jnp.float32),
            jax.ShapeDtypeStruct((TP,), jnp.float32),
            jax.ShapeDtypeStruct((TP,), jnp.float32),
        ],
        mesh=_mesh(),
        scratch_types=[
            pltpu.VMEM((QE,), jnp.int32),
            pltpu.VMEM((QCH, CH), jnp.int32),
            pltpu.VMEM((QCH, CH), jnp.int32),
            pltpu.VMEM((CH, H), jnp.float32),
            pltpu.VMEM((CH, H), jnp.float32),
            pltpu.VMEM((CH,), jnp.float32),
            pltpu.VMEM_SHARED((TP, H), jnp.float32),
            pltpu.VMEM_SHARED((TP,), jnp.float32),
            pltpu.SemaphoreType.DMA,
            pltpu.SemaphoreType.DMA,
        ],
    )(ev0f, us0f, src1d, dst1d, src2d, dst2d, zeros2d, zeros1d)


def _sc2_body(us1f, ev0f, agge0f, uemb, dst1d, src2d, ids_hbm, seqf, dege,
              zeros2d,
              agge1_hbm, e1ids_hbm, e0ids_hbm, ev0ids_hbm, degeids_hbm,
              xseq_hbm,
              idxoff, src_t2, rows0, rows1, ids_t, seq_t, sbuf, degbuf,
              shared, sem0, sem1, semg):
    c = lax.axis_index("c")
    s = lax.axis_index("s")
    coff = c * TP
    w = c * 16 + s

    pltpu.sync_copy(zeros2d.at[pl.ds(s * 640, 640)],
                    shared.at[pl.ds(s * 640, 640)])

    pltpu.sync_copy(ids_hbm, ids_t)

    @pl.when(s == 1)
    def _():
        _offset_idx(ids_t, coff, B)
        pltpu.async_copy(agge0f.at[ids_t], rows0, semg).wait()
        pltpu.sync_copy(rows0, e0ids_hbm.at[pl.ds(c * B, B)])

    @pl.when(s == 2)
    def _():
        _offset_idx(ids_t, coff, B)
        pltpu.async_copy(ev0f.at[ids_t], rows0, semg).wait()
        pltpu.sync_copy(rows0, ev0ids_hbm.at[pl.ds(c * B, B)])

    @pl.when((s == 3) & (c == 0))
    def _():
        pltpu.async_copy(dege.at[ids_t], degbuf, semg).wait()
        pltpu.sync_copy(degbuf, degeids_hbm)

    pltpu.sync_copy(seqf.at[pl.ds(w * 40, 40)], seq_t)
    for r in range(5):
        pltpu.async_copy(uemb.at[seq_t.at[pl.ds(r * 8, 8)]], sbuf, semg).wait()
        pltpu.sync_copy(sbuf, xseq_hbm.at[pl.ds(w * 40 + r * 8, 8)])

    plsc.subcore_barrier()

    _sc_edge_pass(us1f, dst1d, src2d, s, coff, shared,
                  idxoff, src_t2, rows0, rows1, sem0, sem1)
    plsc.subcore_barrier()
    pltpu.sync_copy(shared.at[pl.ds(s * 640, 640)],
                    agge1_hbm.at[pl.ds(coff + s * 640, 640)])
    plsc.subcore_barrier()

    @pl.when(s == 0)
    def _():
        _offset_idx(ids_t, coff, B)
        pltpu.async_copy(agge1_hbm.at[ids_t], rows0, semg).wait()
        pltpu.sync_copy(rows0, e1ids_hbm.at[pl.ds(c * B, B)])


def _sc2(us1f, ev0f, agge0f, uemb, dst1d, src2d, ids, seqf, dege, zeros2d):
    return pl.kernel(
        _sc2_body,
        out_type=[
            jax.ShapeDtypeStruct((2 * TP, H), jnp.float32),
            jax.ShapeDtypeStruct((2 * B, H), jnp.float32),
            jax.ShapeDtypeStruct((2 * B, H), jnp.float32),
            jax.ShapeDtypeStruct((2 * B, H), jnp.float32),
            jax.ShapeDtypeStruct((B,), jnp.float32),
            jax.ShapeDtypeStruct((B * L, D), jnp.float32),
        ],
        mesh=_mesh(),
        scratch_types=[
            pltpu.VMEM((QE,), jnp.int32),
            pltpu.VMEM((QCH, CH), jnp.int32),
            pltpu.VMEM((CH, H), jnp.float32),
            pltpu.VMEM((CH, H), jnp.float32),
            pltpu.VMEM((B,), jnp.int32),
            pltpu.VMEM((40,), jnp.int32),
            pltpu.VMEM((8, D), jnp.float32),
            pltpu.VMEM((B,), jnp.float32),
            pltpu.VMEM_SHARED((TP, H), jnp.float32),
            pltpu.SemaphoreType.DMA,
            pltpu.SemaphoreType.DMA,
            pltpu.SemaphoreType.DMA,
        ],
    )(us1f, ev0f, agge0f, uemb, dst1d, src2d, ids, seqf, dege, zeros2d)


def _head_body(e0_ref, ev0_ref, e1_ref, deg_ref, w_ref, b_ref, hx_ref):
    r = 1.0 / (deg_ref[...] + 1.0)
    ne0 = (e0_ref[...] + ev0_ref[...]) * r
    ev1 = jnp.dot(ne0, w_ref[...],
                  preferred_element_type=jnp.float32) + b_ref[...]
    hx_ref[...] = (e1_ref[...] + ev1) * r


def _head(e0ids, ev0ids, e1ids, degeids_col, we1_t, be1):
    return pl.pallas_call(
        _head_body,
        out_shape=jax.ShapeDtypeStruct((B, D), jnp.float32),
    )(e0ids, ev0ids, e1ids, degeids_col, we1_t, be1)


def _gru_body(seq_ref, hx_ref, wih_ref, whh_ref, bih_ref, bhh_ref,
              out_ref, h_scr):
    t = pl.program_id(0)

    @pl.when(t == 0)
    def _():
        h_scr[...] = hx_ref[...]

    x = seq_ref[0]
    h = h_scr[...]
    gi = jnp.dot(x, wih_ref[...],
                 preferred_element_type=jnp.float32) + bih_ref[...]
    gh = jnp.dot(h, whh_ref[...],
                 preferred_element_type=jnp.float32) + bhh_ref[...]
    r = jax.nn.sigmoid(gi[:, :D] + gh[:, :D])
    z = jax.nn.sigmoid(gi[:, D:2 * D] + gh[:, D:2 * D])
    n = jnp.tanh(gi[:, 2 * D:] + r * gh[:, 2 * D:])
    hn = (1.0 - z) * n + z * h
    h_scr[...] = hn
    out_ref[0] = hn


def _gru(seq_in, hx, wih_t, whh_t, bih, bhh):
    return pl.pallas_call(
        _gru_body,
        grid=(L + 1,),
        in_specs=[
            pl.BlockSpec((1, B, D), lambda t: (t, 0, 0)),
            pl.BlockSpec((B, D), lambda t: (0, 0)),
            pl.BlockSpec((D, 3 * D), lambda t: (0, 0)),
            pl.BlockSpec((D, 3 * D), lambda t: (0, 0)),
            pl.BlockSpec((1, 3 * D), lambda t: (0, 0)),
            pl.BlockSpec((1, 3 * D), lambda t: (0, 0)),
        ],
        out_specs=pl.BlockSpec((1, B, D), lambda t: (t, 0, 0)),
        out_shape=jax.ShapeDtypeStruct((TPAD, B, D), jnp.float32),
        scratch_shapes=[pltpu.VMEM((B, D), jnp.float32)],
    )(seq_in, hx, wih_t, whh_t, bih, bhh)


def _logits_body(og_ref, w_ref, b_ref, seq_ref, out_ref):
    bt = 8
    rows = bt * TPAD
    x = jnp.reshape(og_ref[...], (rows, D))
    lg = jnp.dot(x, w_ref[...],
                 preferred_element_type=jnp.float32) + b_ref[...]
    sq = seq_ref[...]
    vio = lax.broadcasted_iota(jnp.int32, (bt, VP), 1)
    lo = jnp.full((bt, VP), -1, jnp.int32)
    for j in range(L):
        lo = jnp.where(sq[:, j:j + 1] == vio, j, lo)
    lo3 = jnp.reshape(jnp.broadcast_to(lo[:, None, :], (bt, TPAD, VP)),
                      (rows, VP))
    tv = lax.broadcasted_iota(jnp.int32, (rows, 1), 0) % TPAD
    keep = (lo3 >= tv) | (tv >= L - 1)
    lg = jnp.where(keep, lg, lg + NEG)
    m = jnp.max(lg, axis=1, keepdims=True)
    e = jnp.exp(lg - m)
    ssum = jnp.sum(e, axis=1, keepdims=True)
    out_ref[...] = jnp.reshape(e / ssum, (bt, TPAD, VP))


def _logits(og_b, w_lin_t, b_lin_p, sequence):
    bt = 8
    return pl.pallas_call(
        _logits_body,
        grid=(B // bt,),
        in_specs=[
            pl.BlockSpec((bt, TPAD, D), lambda r: (r, 0, 0)),
            pl.BlockSpec((D, VP), lambda r: (0, 0)),
            pl.BlockSpec((1, VP), lambda r: (0, 0)),
            pl.BlockSpec((bt, L), lambda r: (r, 0)),
        ],
        out_specs=pl.BlockSpec((bt, TPAD, VP), lambda r: (r, 0, 0)),
        out_shape=jax.ShapeDtypeStruct((B, TPAD, VP), jnp.float32),
    )(og_b, w_lin_t, b_lin_p, sequence)


def kernel(x_user, x_event, W_u0, b_u0, W_e0, b_e0, W_u1, b_u1, W_e1, b_e1,
           W_ih, W_hh, b_ih, b_hh, W_lin, b_lin, start, user_embedding,
           edge_index, ids, sequence, batch_size):
    f32 = jnp.float32
    src = edge_index[0]
    dst = edge_index[1]
    padlen = EP - E
    dump = jnp.full((padlen,), TP - 1, jnp.int32)
    src1d = jnp.concatenate([src, dump])
    dst1d = jnp.concatenate([dst, dump])
    src2d = src1d.reshape(EP // CH, CH)
    dst2d = dst1d.reshape(EP // CH, CH)
    zeros2d = jnp.zeros((TP, H), f32)
    zeros1d = jnp.zeros((TP,), f32)

    us0s, ev0s = _dense0(x_user, x_event,
                         W_u0.T.astype(f32), W_e0.T.astype(f32),
                         b_u0.reshape(1, D).astype(f32),
                         b_e0.reshape(1, D).astype(f32))
    us0f = us0s.reshape(2 * TP, H)
    ev0f = ev0s.reshape(2 * TP, H)

    aggu_f, agge0f, deg_u, deg_e = _sc1(ev0f, us0f, src1d, dst1d,
                                        src2d, dst2d, zeros2d, zeros1d)

    us1s = _dense1(aggu_f.reshape(2, TP, H), us0s, deg_u.reshape(TP, 1),
                   W_u1.T.astype(f32), b_u1.reshape(1, D).astype(f32))
    us1f = us1s.reshape(2 * TP, H)

    seqf = sequence.T.reshape(B * L)
    (_agge1f, e1ids, e0ids, ev0ids, degeids, xseq) = _sc2(
        us1f, ev0f, agge0f, user_embedding.astype(f32), dst1d, src2d,
        ids, seqf, deg_e, zeros2d)

    e0c = jnp.concatenate([e0ids[:B], e0ids[B:]], axis=1)
    e1c = jnp.concatenate([e1ids[:B], e1ids[B:]], axis=1)
    ev0c = jnp.concatenate([ev0ids[:B], ev0ids[B:]], axis=1)

    hx = _head(e0c, ev0c, e1c, degeids.reshape(B, 1),
               W_e1.T.astype(f32), b_e1.reshape(1, D).astype(f32))

    seq_in = jnp.concatenate(
        [jnp.broadcast_to(start.astype(f32), (1, B, D)),
         xseq.reshape(L, B, D)], axis=0)
    outg = _gru(seq_in, hx, W_ih.T.astype(f32), W_hh.T.astype(f32),
                b_ih.reshape(1, 3 * D).astype(f32),
                b_hh.reshape(1, 3 * D).astype(f32))
    og_b = jnp.transpose(outg, (1, 0, 2))

    w_lin_t = jnp.zeros((D, VP), f32).at[:, :NU].set(W_lin.T.astype(f32))
    b_lin_p = jnp.full((1, VP), NEG, f32).at[0, :NU].set(b_lin.astype(f32))
    probs_p = _logits(og_b, w_lin_t, b_lin_p, sequence)
    return lax.slice(probs_p, (0, 0, 0), (B, L + 1, NU))


# speedup vs baseline: 3.8251x; 1.7742x over previous
import functools

import jax
import jax.numpy as jnp
from jax import lax
from jax.experimental import pallas as pl
from jax.experimental.pallas import tpu as pltpu
from jax.experimental.pallas import tpu_sc as plsc

NU = 10000
NE = 10000
D = 256
E = 160000
B = 64
L = 20
TP = 10240
EP = 163840
EPT = EP // 16
CH = 64
NCH = EPT // CH
H = 128
TPAD = 24
VP = 10240
NEG = -1e9

def _mesh():
    return plsc.VectorSubcoreMesh(core_axis_name="c", subcore_axis_name="s",
                                  num_cores=2, num_subcores=16)


def _dense0_body(xu_ref, xe_ref, wu_ref, we_ref, bu_ref, be_ref,
                 us_ref, ev_ref):
    us = jnp.dot(xu_ref[...], wu_ref[...],
                 preferred_element_type=jnp.float32) + bu_ref[...]
    ev = jnp.dot(xe_ref[...], we_ref[...],
                 preferred_element_type=jnp.float32) + be_ref[...]
    us_ref[0] = us[:, :H]
    us_ref[1] = us[:, H:]
    ev_ref[0] = ev[:, :H]
    ev_ref[1] = ev[:, H:]


def _dense0(x_user, x_event, wu_t, we_t, bu, be):
    blk = 1000
    return pl.pallas_call(
        _dense0_body,
        grid=(NU // blk,),
        in_specs=[
            pl.BlockSpec((blk, D), lambda r: (r, 0)),
            pl.BlockSpec((blk, D), lambda r: (r, 0)),
            pl.BlockSpec((D, D), lambda r: (0, 0)),
            pl.BlockSpec((D, D), lambda r: (0, 0)),
            pl.BlockSpec((1, D), lambda r: (0, 0)),
            pl.BlockSpec((1, D), lambda r: (0, 0)),
        ],
        out_specs=[
            pl.BlockSpec((2, blk, H), lambda r: (0, r, 0)),
            pl.BlockSpec((2, blk, H), lambda r: (0, r, 0)),
        ],
        out_shape=[
            jax.ShapeDtypeStruct((2, TP, H), jnp.float32),
            jax.ShapeDtypeStruct((2, TP, H), jnp.float32),
        ],
    )(x_user, x_event, wu_t, we_t, bu, be)


def _dense1_body(agg_ref, us0_ref, deg_ref, w_ref, b_ref, out_ref):
    agg = jnp.concatenate([agg_ref[0], agg_ref[1]], axis=1)
    us0 = jnp.concatenate([us0_ref[0], us0_ref[1]], axis=1)
    new_u = (agg + us0) * (1.0 / (deg_ref[...] + 1.0))
    us1 = jnp.dot(new_u, w_ref[...],
                  preferred_element_type=jnp.float32) + b_ref[...]
    out_ref[0] = us1[:, :H]
    out_ref[1] = us1[:, H:]


def _dense1(agg_u, us0s, deg_u_col, w1_t, b1):
    blk = 1000
    return pl.pallas_call(
        _dense1_body,
        grid=(NU // blk,),
        in_specs=[
            pl.BlockSpec((2, blk, H), lambda r: (0, r, 0)),
            pl.BlockSpec((2, blk, H), lambda r: (0, r, 0)),
            pl.BlockSpec((blk, 1), lambda r: (r, 0)),
            pl.BlockSpec((D, D), lambda r: (0, 0)),
            pl.BlockSpec((1, D), lambda r: (0, 0)),
        ],
        out_specs=pl.BlockSpec((2, blk, H), lambda r: (0, r, 0)),
        out_shape=jax.ShapeDtypeStruct((2, TP, H), jnp.float32),
    )(agg_u, us0s, deg_u_col, w1_t, b1)


NQ = 4
QCH = NCH // NQ
QE = EPT // NQ


def _sc_edge_pass(table_hbm, gsrc1d, widx2d, s, coff, shared,
                  idxoff, widx_t2, rows0, rows1, sem0, sem1, hist=None):
    def g_start(j, buf, sem):
        off = pl.multiple_of(j * CH, CH)
        pltpu.async_copy(table_hbm.at[idxoff.at[pl.ds(off, CH)]], buf, sem)

    def g_wait(buf, sem):
        pltpu.make_async_copy(
            table_hbm.at[idxoff.at[pl.ds(0, CH)]], buf, sem).wait()

    for q in range(NQ):
        pltpu.sync_copy(gsrc1d.at[pl.ds(s * EPT + q * QE, QE)], idxoff)
        _offset_idx(idxoff, coff, QE)
        pltpu.sync_copy(widx2d.at[pl.ds(s * NCH + q * QCH, QCH)], widx_t2)
        if hist is not None:
            c, src2d_hbm, hsrc_t2, hist_ref, ones = hist
            pltpu.sync_copy(src2d_hbm.at[pl.ds(s * NCH + q * QCH, QCH)],
                            hsrc_t2)

        def do_chunk(j, buf):
            pltpu.sync_copy(buf, shared.at[widx_t2.at[j]], add=True)
            if hist is not None:
                @pl.when(c == 0)
                def _():
                    pltpu.sync_copy(ones, hist_ref.at[widx_t2.at[j]],
                                    add=True)

                @pl.when(c == 1)
                def _():
                    pltpu.sync_copy(ones, hist_ref.at[hsrc_t2.at[j]],
                                    add=True)

        g_start(0, rows0, sem0)

        def body(i, carry):
            j0 = i * 2
            j1 = j0 + 1
            g_wait(rows0, sem0)
            g_start(j1, rows1, sem1)
            do_chunk(j0, rows0)
            g_wait(rows1, sem1)

            @pl.when(i < QCH // 2 - 1)
            def _():
                g_start(j0 + 2, rows0, sem0)

            do_chunk(j1, rows1)
            return carry

        lax.fori_loop(0, QCH // 2, body, 0)


def _offset_idx(idx_ref, coff, n):
    def body(i, carry):
        off = pl.multiple_of(i * 16, 16)
        idx_ref[pl.ds(off, 16)] = idx_ref[pl.ds(off, 16)] + coff
        return carry

    lax.fori_loop(0, n // 16, body, 0)


REG = EP // 32


def _sc0_body(src1d, dst1d, ids_hbm,
              msrc_hbm, mdst_hbm, cnts_hbm,
              src_st, dst_st, ismem, osrc, odst, ids_t, cntbuf):
    c = lax.axis_index("c")
    s = lax.axis_index("s")
    w = c * 16 + s
    i32 = jnp.int32

    pltpu.sync_copy(src1d.at[pl.ds(w * REG, REG)], src_st)
    pltpu.sync_copy(dst1d.at[pl.ds(w * REG, REG)], dst_st)
    pltpu.sync_copy(ids_hbm, ids_t)

    def zero(i, carry):
        ismem[pl.ds(pl.multiple_of(i * 16, 16), 16)] = jnp.zeros((16,), i32)
        return carry

    lax.fori_loop(0, TP // 16, zero, 0)
    one16 = jnp.ones((16,), i32)
    mtrue = jnp.ones((16,), jnp.bool_)
    for k in range(B // 16):
        iv = ids_t[pl.ds(k * 16, 16)]
        plsc.store_scatter(ismem, [iv], one16, mask=mtrue)

    def scan(i, cnt):
        off = pl.multiple_of(i * 16, 16)
        sv = src_st[pl.ds(off, 16)]
        dv = dst_st[pl.ds(off, 16)]
        m = plsc.load_gather(ismem, [sv]) > 0
        mi = jnp.where(m, 1, 0).astype(i32)
        pos = cnt + plsc.cumsum(mi) - 1
        plsc.store_scatter(osrc, [pos], sv, mask=m)
        plsc.store_scatter(odst, [pos], dv, mask=m)
        return cnt + plsc.all_reduce_population_count(m)

    cnt = lax.fori_loop(0, REG // 16, scan, jnp.zeros((16,), i32))
    padc = ((cnt + (CH - 1)) // CH) * CH
    lane = lax.iota(i32, 16)
    dumpv = jnp.full((16,), TP - 1, i32)
    zerov = jnp.zeros((16,), i32)
    for k in range(CH // 16):
        pos = cnt + k * 16 + lane
        mf = pos < padc
        plsc.store_scatter(osrc, [pos], dumpv, mask=mf)
        plsc.store_scatter(odst, [pos], zerov, mask=mf)
    cntbuf[...] = padc
    pltpu.sync_copy(cntbuf, cnts_hbm.at[w])
    pltpu.sync_copy(osrc, msrc_hbm.at[pl.ds(w * REG, REG)])
    pltpu.sync_copy(odst, mdst_hbm.at[pl.ds(w * REG, REG)])


def _sc0(src1d, dst1d, ids):
    return pl.kernel(
        _sc0_body,
        out_type=[
            jax.ShapeDtypeStruct((EP,), jnp.int32),
            jax.ShapeDtypeStruct((EP,), jnp.int32),
            jax.ShapeDtypeStruct((32, 16), jnp.int32),
        ],
        mesh=_mesh(),
        compiler_params=pltpu.CompilerParams(needs_layout_passes=False),
        scratch_types=[
            pltpu.VMEM((REG,), jnp.int32),
            pltpu.VMEM((REG,), jnp.int32),
            pltpu.VMEM((TP,), jnp.int32),
            pltpu.VMEM((REG,), jnp.int32),
            pltpu.VMEM((REG,), jnp.int32),
            pltpu.VMEM((B,), jnp.int32),
            pltpu.VMEM((16,), jnp.int32),
        ],
    )(src1d, dst1d, ids)


def _sc_matched_pass(table_hbm, mdst1d, msrc2d, cnts2, s, coff, shared,
                     cidx, wrow, rowsb, cbuf, semg):
    for rr in range(2):
        rg = 2 * s + rr
        pltpu.sync_copy(cnts2.at[rg], cbuf)
        nch = jnp.max(cbuf[...], axis=0) // CH

        def body(j, carry):
            base = rg * REG + j * CH
            pltpu.sync_copy(mdst1d.at[pl.ds(base, CH)], cidx)
            _offset_idx(cidx, coff, CH)
            pltpu.sync_copy(msrc2d.at[pl.ds(rg * (REG // CH) + j, 1)], wrow)
            pltpu.async_copy(table_hbm.at[cidx], rowsb, semg).wait()
            pltpu.sync_copy(rowsb, shared.at[wrow.at[0]], add=True)
            return carry

        lax.fori_loop(0, nch, body, 0)


def _sc1_body(ev0f, us0f, src1d, src2d, dst2d, mdst1d, msrc2d, cnts2,
              zeros2d, zeros1d,
              aggu_hbm, agge0_hbm, degu_hbm, dege_hbm,
              idxoff, widx_t2, hsrc_t2, rows0, rows1, ones, cidx, wrow, cbuf,
              shared, hist, sem0, sem1):
    c = lax.axis_index("c")
    s = lax.axis_index("s")
    coff = c * TP

    for li in range(CH // 16):
        ones[pl.ds(li * 16, 16)] = jnp.full((16,), 1.0, jnp.float32)

    pltpu.sync_copy(zeros2d.at[pl.ds(s * 640, 640)],
                    shared.at[pl.ds(s * 640, 640)])

    @pl.when(s == 0)
    def _():
        pltpu.sync_copy(zeros1d, hist)

    plsc.subcore_barrier()

    _sc_edge_pass(ev0f, src1d, dst2d, s, coff, shared,
                  idxoff, widx_t2, rows0, rows1, sem0, sem1,
                  hist=(c, src2d, hsrc_t2, hist, ones))
    plsc.subcore_barrier()
    pltpu.sync_copy(shared.at[pl.ds(s * 640, 640)],
                    aggu_hbm.at[pl.ds(coff + s * 640, 640)])

    @pl.when((c == 0) & (s == 0))
    def _():
        pltpu.sync_copy(hist, degu_hbm)

    @pl.when((c == 1) & (s == 0))
    def _():
        pltpu.sync_copy(hist, dege_hbm)

    plsc.subcore_barrier()
    pltpu.sync_copy(zeros2d.at[pl.ds(s * 640, 640)],
                    shared.at[pl.ds(s * 640, 640)])
    plsc.subcore_barrier()

    _sc_matched_pass(us0f, mdst1d, msrc2d, cnts2, s, coff, shared,
                     cidx, wrow, rows0, cbuf, sem0)
    plsc.subcore_barrier()
    pltpu.sync_copy(shared.at[pl.ds(s * 640, 640)],
                    agge0_hbm.at[pl.ds(coff + s * 640, 640)])


def _sc1(ev0f, us0f, src1d, src2d, dst2d, mdst1d, msrc2d, cnts2,
         zeros2d, zeros1d):
    return pl.kernel(
        _sc1_body,
        out_type=[
            jax.ShapeDtypeStruct((2 * TP, H), jnp.float32),
            jax.ShapeDtypeStruct((2 * TP, H), jnp.float32),
            jax.ShapeDtypeStruct((TP,), jnp.float32),
            jax.ShapeDtypeStruct((TP,), jnp.float32),
        ],
        mesh=_mesh(),
        compiler_params=pltpu.CompilerParams(needs_layout_passes=False),
        scratch_types=[
            pltpu.VMEM((QE,), jnp.int32),
            pltpu.VMEM((QCH, CH), jnp.int32),
            pltpu.VMEM((QCH, CH), jnp.int32),
            pltpu.VMEM((CH, H), jnp.float32),
            pltpu.VMEM((CH, H), jnp.float32),
            pltpu.VMEM((CH,), jnp.float32),
            pltpu.VMEM((CH,), jnp.int32),
            pltpu.VMEM((1, CH), jnp.int32),
            pltpu.VMEM((16,), jnp.int32),
            pltpu.VMEM_SHARED((TP, H), jnp.float32),
            pltpu.VMEM_SHARED((TP,), jnp.float32),
            pltpu.SemaphoreType.DMA,
            pltpu.SemaphoreType.DMA,
        ],
    )(ev0f, us0f, src1d, src2d, dst2d, mdst1d, msrc2d, cnts2,
      zeros2d, zeros1d)


def _sc2_body(us1f, ev0f, agge0f, uemb, mdst1d, msrc2d, cnts2, ids_hbm,
              seqf, dege, zeros2d,
              agge1_hbm, e1ids_hbm, e0ids_hbm, ev0ids_hbm, degeids_hbm,
              xseq_hbm,
              rows0, ids_t, seq_t, sbuf, degbuf, cidx, wrow, cbuf,
              shared, semg):
    c = lax.axis_index("c")
    s = lax.axis_index("s")
    coff = c * TP
    w = c * 16 + s

    pltpu.sync_copy(zeros2d.at[pl.ds(s * 640, 640)],
                    shared.at[pl.ds(s * 640, 640)])

    pltpu.sync_copy(ids_hbm, ids_t)

    @pl.when(s == 1)
    def _():
        _offset_idx(ids_t, coff, B)
        pltpu.async_copy(agge0f.at[ids_t], rows0, semg).wait()
        pltpu.sync_copy(rows0, e0ids_hbm.at[pl.ds(c * B, B)])

    @pl.when(s == 2)
    def _():
        _offset_idx(ids_t, coff, B)
        pltpu.async_copy(ev0f.at[ids_t], rows0, semg).wait()
        pltpu.sync_copy(rows0, ev0ids_hbm.at[pl.ds(c * B, B)])

    @pl.when((s == 3) & (c == 0))
    def _():
        pltpu.async_copy(dege.at[ids_t], degbuf, semg).wait()
        pltpu.sync_copy(degbuf, degeids_hbm)

    pltpu.sync_copy(seqf.at[pl.ds(w * 40, 40)], seq_t)
    for r in range(5):
        pltpu.async_copy(uemb.at[seq_t.at[pl.ds(r * 8, 8)]], sbuf, semg).wait()
        pltpu.sync_copy(sbuf, xseq_hbm.at[pl.ds(w * 40 + r * 8, 8)])

    plsc.subcore_barrier()

    _sc_matched_pass(us1f, mdst1d, msrc2d, cnts2, s, coff, shared,
                     cidx, wrow, rows0, cbuf, semg)
    plsc.subcore_barrier()
    pltpu.sync_copy(shared.at[pl.ds(s * 640, 640)],
                    agge1_hbm.at[pl.ds(coff + s * 640, 640)])
    plsc.subcore_barrier()

    @pl.when(s == 0)
    def _():
        _offset_idx(ids_t, coff, B)
        pltpu.async_copy(agge1_hbm.at[ids_t], rows0, semg).wait()
        pltpu.sync_copy(rows0, e1ids_hbm.at[pl.ds(c * B, B)])


def _sc2(us1f, ev0f, agge0f, uemb, mdst1d, msrc2d, cnts2, ids, seqf, dege,
         zeros2d):
    return pl.kernel(
        _sc2_body,
        out_type=[
            jax.ShapeDtypeStruct((2 * TP, H), jnp.float32),
            jax.ShapeDtypeStruct((2 * B, H), jnp.float32),
            jax.ShapeDtypeStruct((2 * B, H), jnp.float32),
            jax.ShapeDtypeStruct((2 * B, H), jnp.float32),
            jax.ShapeDtypeStruct((B,), jnp.float32),
            jax.ShapeDtypeStruct((B * L, D), jnp.float32),
        ],
        mesh=_mesh(),
        compiler_params=pltpu.CompilerParams(needs_layout_passes=False),
        scratch_types=[
            pltpu.VMEM((CH, H), jnp.float32),
            pltpu.VMEM((B,), jnp.int32),
            pltpu.VMEM((40,), jnp.int32),
            pltpu.VMEM((8, D), jnp.float32),
            pltpu.VMEM((B,), jnp.float32),
            pltpu.VMEM((CH,), jnp.int32),
            pltpu.VMEM((1, CH), jnp.int32),
            pltpu.VMEM((16,), jnp.int32),
            pltpu.VMEM_SHARED((TP, H), jnp.float32),
            pltpu.SemaphoreType.DMA,
        ],
    )(us1f, ev0f, agge0f, uemb, mdst1d, msrc2d, cnts2, ids, seqf, dege,
      zeros2d)


def _head_body(e0_ref, ev0_ref, e1_ref, deg_ref, w_ref, b_ref, hx_ref):
    r = 1.0 / (deg_ref[...] + 1.0)
    ne0 = (e0_ref[...] + ev0_ref[...]) * r
    ev1 = jnp.dot(ne0, w_ref[...],
                  preferred_element_type=jnp.float32) + b_ref[...]
    hx_ref[...] = (e1_ref[...] + ev1) * r


def _head(e0ids, ev0ids, e1ids, degeids_col, we1_t, be1):
    return pl.pallas_call(
        _head_body,
        out_shape=jax.ShapeDtypeStruct((B, D), jnp.float32),
    )(e0ids, ev0ids, e1ids, degeids_col, we1_t, be1)


def _gru_body(seq_ref, hx_ref, wih_ref, whh_ref, bih_ref, bhh_ref,
              out_ref, h_scr):
    t = pl.program_id(0)

    @pl.when(t == 0)
    def _():
        h_scr[...] = hx_ref[...]

    x = seq_ref[0]
    h = h_scr[...]
    gi = jnp.dot(x, wih_ref[...],
                 preferred_element_type=jnp.float32) + bih_ref[...]
    gh = jnp.dot(h, whh_ref[...],
                 preferred_element_type=jnp.float32) + bhh_ref[...]
    r = jax.nn.sigmoid(gi[:, :D] + gh[:, :D])
    z = jax.nn.sigmoid(gi[:, D:2 * D] + gh[:, D:2 * D])
    n = jnp.tanh(gi[:, 2 * D:] + r * gh[:, 2 * D:])
    hn = (1.0 - z) * n + z * h
    h_scr[...] = hn
    out_ref[0] = hn


def _gru(seq_in, hx, wih_t, whh_t, bih, bhh):
    return pl.pallas_call(
        _gru_body,
        grid=(L + 1,),
        in_specs=[
            pl.BlockSpec((1, B, D), lambda t: (t, 0, 0)),
            pl.BlockSpec((B, D), lambda t: (0, 0)),
            pl.BlockSpec((D, 3 * D), lambda t: (0, 0)),
            pl.BlockSpec((D, 3 * D), lambda t: (0, 0)),
            pl.BlockSpec((1, 3 * D), lambda t: (0, 0)),
            pl.BlockSpec((1, 3 * D), lambda t: (0, 0)),
        ],
        out_specs=pl.BlockSpec((1, B, D), lambda t: (t, 0, 0)),
        out_shape=jax.ShapeDtypeStruct((TPAD, B, D), jnp.float32),
        scratch_shapes=[pltpu.VMEM((B, D), jnp.float32)],
    )(seq_in, hx, wih_t, whh_t, bih, bhh)


def _logits_body(og_ref, w_ref, b_ref, seq_ref, out_ref):
    bt = 8
    rows = bt * TPAD
    x = jnp.reshape(og_ref[...], (rows, D))
    lg = jnp.dot(x, w_ref[...],
                 preferred_element_type=jnp.float32) + b_ref[...]
    sq = seq_ref[...]
    vio = lax.broadcasted_iota(jnp.int32, (bt, VP), 1)
    lo = jnp.full((bt, VP), -1, jnp.int32)
    for j in range(L):
        lo = jnp.where(sq[:, j:j + 1] == vio, j, lo)
    lo3 = jnp.reshape(jnp.broadcast_to(lo[:, None, :], (bt, TPAD, VP)),
                      (rows, VP))
    tv = lax.broadcasted_iota(jnp.int32, (rows, 1), 0) % TPAD
    keep = (lo3 >= tv) | (tv >= L - 1)
    lg = jnp.where(keep, lg, lg + NEG)
    m = jnp.max(lg, axis=1, keepdims=True)
    e = jnp.exp(lg - m)
    ssum = jnp.sum(e, axis=1, keepdims=True)
    out_ref[...] = jnp.reshape(e / ssum, (bt, TPAD, VP))


def _logits(og_b, w_lin_t, b_lin_p, sequence):
    bt = 8
    return pl.pallas_call(
        _logits_body,
        grid=(B // bt,),
        in_specs=[
            pl.BlockSpec((bt, TPAD, D), lambda r: (r, 0, 0)),
            pl.BlockSpec((D, VP), lambda r: (0, 0)),
            pl.BlockSpec((1, VP), lambda r: (0, 0)),
            pl.BlockSpec((bt, L), lambda r: (r, 0)),
        ],
        out_specs=pl.BlockSpec((bt, TPAD, VP), lambda r: (r, 0, 0)),
        out_shape=jax.ShapeDtypeStruct((B, TPAD, VP), jnp.float32),
    )(og_b, w_lin_t, b_lin_p, sequence)


def kernel(x_user, x_event, W_u0, b_u0, W_e0, b_e0, W_u1, b_u1, W_e1, b_e1,
           W_ih, W_hh, b_ih, b_hh, W_lin, b_lin, start, user_embedding,
           edge_index, ids, sequence, batch_size):
    f32 = jnp.float32
    src = edge_index[0]
    dst = edge_index[1]
    padlen = EP - E
    dump = jnp.full((padlen,), TP - 1, jnp.int32)
    src1d = jnp.concatenate([src, dump])
    dst1d = jnp.concatenate([dst, dump])
    src2d = src1d.reshape(EP // CH, CH)
    dst2d = dst1d.reshape(EP // CH, CH)
    zeros2d = jnp.zeros((TP, H), f32)
    zeros1d = jnp.zeros((TP,), f32)

    us0s, ev0s = _dense0(x_user, x_event,
                         W_u0.T.astype(f32), W_e0.T.astype(f32),
                         b_u0.reshape(1, D).astype(f32),
                         b_e0.reshape(1, D).astype(f32))
    us0f = us0s.reshape(2 * TP, H)
    ev0f = ev0s.reshape(2 * TP, H)

    msrc, mdst, cnts2 = _sc0(src1d, dst1d, ids)
    msrc2d = msrc.reshape(EP // CH, CH)

    aggu_f, agge0f, deg_u, deg_e = _sc1(ev0f, us0f, src1d, src2d, dst2d,
                                        mdst, msrc2d, cnts2,
                                        zeros2d, zeros1d)

    us1s = _dense1(aggu_f.reshape(2, TP, H), us0s, deg_u.reshape(TP, 1),
                   W_u1.T.astype(f32), b_u1.reshape(1, D).astype(f32))
    us1f = us1s.reshape(2 * TP, H)

    seqf = sequence.T.reshape(B * L)
    (_agge1f, e1ids, e0ids, ev0ids, degeids, xseq) = _sc2(
        us1f, ev0f, agge0f, user_embedding.astype(f32), mdst, msrc2d, cnts2,
        ids, seqf, deg_e, zeros2d)

    e0c = jnp.concatenate([e0ids[:B], e0ids[B:]], axis=1)
    e1c = jnp.concatenate([e1ids[:B], e1ids[B:]], axis=1)
    ev0c = jnp.concatenate([ev0ids[:B], ev0ids[B:]], axis=1)

    hx = _head(e0c, ev0c, e1c, degeids.reshape(B, 1),
               W_e1.T.astype(f32), b_e1.reshape(1, D).astype(f32))

    seq_in = jnp.concatenate(
        [jnp.broadcast_to(start.astype(f32), (1, B, D)),
         xseq.reshape(L, B, D)], axis=0)
    outg = _gru(seq_in, hx, W_ih.T.astype(f32), W_hh.T.astype(f32),
                b_ih.reshape(1, 3 * D).astype(f32),
                b_hh.reshape(1, 3 * D).astype(f32))
    og_b = jnp.transpose(outg, (1, 0, 2))

    w_lin_t = jnp.zeros((D, VP), f32).at[:, :NU].set(W_lin.T.astype(f32))
    b_lin_p = jnp.full((1, VP), NEG, f32).at[0, :NU].set(b_lin.astype(f32))
    probs_p = _logits(og_b, w_lin_t, b_lin_p, sequence)
    return lax.slice(probs_p, (0, 0, 0), (B, L + 1, NU))
